# Initial kernel scaffold; baseline (speedup 1.0000x reference)
import jax
import jax.numpy as jnp
from jax.experimental import pallas as pl


def kernel(output_ids, parent_ids, out_seq_lens, end_id):
    raise NotImplementedError("write your pallas kernel here")


# speedup vs baseline: 192.4620x; 192.4620x over previous
import functools

import jax
import jax.numpy as jnp
from jax import lax
from jax.experimental import pallas as pl
from jax.experimental.pallas import tpu as pltpu
from jax.experimental.pallas import tpu_sc as plsc

T, B, K = 2048, 32, 8
LANES = 16


def _sc_backtrack_body(step_hbm, par_hbm, osl_hbm, raw_hbm,
                       step_v, par_v, out_v, osl_v):
    c = lax.axis_index("c")
    s = lax.axis_index("s")
    b = s * 2 + c

    pltpu.sync_copy(step_hbm.at[b], step_v)
    pltpu.sync_copy(par_hbm.at[b], par_v)
    pltpu.sync_copy(osl_hbm.at[pl.ds(b * K, K)], osl_v.at[pl.ds(0, K)])

    lane = lax.iota(jnp.int32, LANES)
    kvec = lane & (K - 1)
    osl = osl_v[...]
    L = osl[0]
    for i in range(1, K):
        L = jnp.maximum(L, osl[i])
    L = jnp.minimum(L, T)

    def body(i, parent):
        t = L - 1 - i
        base = jnp.full((LANES,), t * K, dtype=jnp.int32)
        p = jnp.minimum(jnp.maximum(parent, 0), K - 1)
        idx = base + p
        sv = plsc.load_gather(step_v, [idx])
        pv = plsc.load_gather(par_v, [idx])
        plsc.store_scatter(out_v, [kvec * T + t], sv, mask=lane < K)
        return pv

    lax.fori_loop(0, L, body, kvec)
    pltpu.sync_copy(out_v, raw_hbm.at[b])


_sc_backtrack = functools.partial(
    pl.kernel,
    mesh=plsc.VectorSubcoreMesh(core_axis_name="c", subcore_axis_name="s"),
    out_type=jax.ShapeDtypeStruct((B, K * T), jnp.int32),
    compiler_params=pltpu.CompilerParams(needs_layout_passes=False),
    scratch_types=[
        pltpu.VMEM((T * K,), jnp.int32),
        pltpu.VMEM((T * K,), jnp.int32),
        pltpu.VMEM((K * T,), jnp.int32),
        pltpu.VMEM((LANES,), jnp.int32),
    ],
)(_sc_backtrack_body)


def _tc_post_body(end_ref, osl_ref, raw_ref, ids_ref, len_ref):
    end = end_ref[0]
    L = jnp.minimum(jnp.max(osl_ref[...]), T)
    raw = raw_ref[...]
    col = lax.broadcasted_iota(jnp.int32, (K, T), 1)
    cand = jnp.where((raw == end) & (col < L), col, L)
    fe = jnp.min(cand, axis=1, keepdims=True)
    ids_ref[...] = jnp.where(col < fe, raw, end)
    len_ref[...] = fe


def _tc_post(end_arr, osl3, raw):
    return pl.pallas_call(
        _tc_post_body,
        grid=(B,),
        in_specs=[
            pl.BlockSpec(memory_space=pltpu.SMEM),
            pl.BlockSpec((1, 1, K), lambda b: (b, 0, 0)),
            pl.BlockSpec((K, T), lambda b: (b, 0)),
        ],
        out_specs=[
            pl.BlockSpec((K, T), lambda b: (b, 0)),
            pl.BlockSpec((K, 1), lambda b: (b, 0)),
        ],
        out_shape=[
            jax.ShapeDtypeStruct((B * K, T), jnp.int32),
            jax.ShapeDtypeStruct((B * K, 1), jnp.int32),
        ],
    )(end_arr, osl3, raw)


def kernel(output_ids, parent_ids, out_seq_lens, end_id):
    step_r = output_ids.reshape(T, B, K).transpose(1, 0, 2).reshape(B, T * K)
    par_r = parent_ids.reshape(T, B, K).transpose(1, 0, 2).reshape(B, T * K)
    osl = out_seq_lens.astype(jnp.int32)
    raw = _sc_backtrack(step_r, par_r, osl)
    end_arr = jnp.asarray(end_id, jnp.int32).reshape(1)
    ids, lengths = _tc_post(end_arr, osl.reshape(B, 1, K), raw.reshape(B * K, T))
    return ids.reshape(B, K, T), lengths.reshape(B, K)


# speedup vs baseline: 242.3223x; 1.2591x over previous
import functools

import jax
import jax.numpy as jnp
from jax import lax
from jax.experimental import pallas as pl
from jax.experimental.pallas import tpu as pltpu
from jax.experimental.pallas import tpu_sc as plsc

T, B, K = 2048, 32, 8
LANES = 16


def _sc_backtrack_body(step_hbm, par_hbm, osl_hbm, end_hbm, raw_hbm,
                       step_v, par_v, out_v, osl_v, end_v):
    c = lax.axis_index("c")
    s = lax.axis_index("s")
    b = s * 2 + c

    pltpu.sync_copy(step_hbm.at[b], step_v)
    pltpu.sync_copy(par_hbm.at[b], par_v)
    pltpu.sync_copy(osl_hbm.at[pl.ds(b * K, K)], osl_v.at[pl.ds(0, K)])
    pltpu.sync_copy(end_hbm, end_v)

    lane = lax.iota(jnp.int32, LANES)
    kvec = lane & (K - 1)
    osl = osl_v[...]
    L = osl[0]
    for i in range(1, K):
        L = jnp.maximum(L, osl[i])
    L = jnp.minimum(L, T)

    plsc.store_scatter(out_v, [kvec, jnp.full((LANES,), L, jnp.int32)],
                       end_v[...], mask=lane < K)

    def body(i, parent):
        t = L - 1 - i
        base = jnp.full((LANES,), t * K, dtype=jnp.int32)
        p = jnp.minimum(jnp.maximum(parent, 0), K - 1)
        idx = base + p
        sv = plsc.load_gather(step_v, [idx])
        pv = plsc.load_gather(par_v, [idx])
        plsc.store_scatter(out_v, [kvec, jnp.full((LANES,), t, jnp.int32)],
                           sv, mask=lane < K)
        return pv

    lax.fori_loop(0, L, body, kvec)
    pltpu.sync_copy(out_v, raw_hbm.at[pl.ds(b * K, K), :])


_sc_backtrack = functools.partial(
    pl.kernel,
    mesh=plsc.VectorSubcoreMesh(core_axis_name="c", subcore_axis_name="s"),
    out_type=jax.ShapeDtypeStruct((B * K, T + 8), jnp.int32),
    compiler_params=pltpu.CompilerParams(needs_layout_passes=False),
    scratch_types=[
        pltpu.VMEM((T * K,), jnp.int32),
        pltpu.VMEM((T * K,), jnp.int32),
        pltpu.VMEM((K, T + 8), jnp.int32),
        pltpu.VMEM((LANES,), jnp.int32),
        pltpu.VMEM((LANES,), jnp.int32),
    ],
)(_sc_backtrack_body)

RB = 64


def _tc_post_body(end_ref, raw_ref, ids_ref, len_ref):
    end = end_ref[0]
    raw = raw_ref[...]
    col = lax.broadcasted_iota(jnp.int32, (RB, T + 8), 1)
    cand = jnp.where(raw == end, col, T)
    fe = jnp.min(cand, axis=1, keepdims=True)
    ids_ref[...] = jnp.where(col[:, :T] < fe, raw[:, :T], end)
    len_ref[...] = fe


def _tc_post(end_arr, raw):
    return pl.pallas_call(
        _tc_post_body,
        grid=(B * K // RB,),
        in_specs=[
            pl.BlockSpec(memory_space=pltpu.SMEM),
            pl.BlockSpec((RB, T + 8), lambda i: (i, 0)),
        ],
        out_specs=[
            pl.BlockSpec((RB, T), lambda i: (i, 0)),
            pl.BlockSpec((RB, 1), lambda i: (i, 0)),
        ],
        out_shape=[
            jax.ShapeDtypeStruct((B * K, T), jnp.int32),
            jax.ShapeDtypeStruct((B * K, 1), jnp.int32),
        ],
    )(end_arr, raw)


def kernel(output_ids, parent_ids, out_seq_lens, end_id):
    step_r = output_ids.reshape(T, B, K).transpose(1, 0, 2).reshape(B, T * K)
    par_r = parent_ids.reshape(T, B, K).transpose(1, 0, 2).reshape(B, T * K)
    osl = out_seq_lens.astype(jnp.int32)
    end_i32 = jnp.asarray(end_id, jnp.int32)
    end_pad = jnp.full((LANES,), end_i32, jnp.int32)
    raw = _sc_backtrack(step_r, par_r, osl, end_pad)
    ids, lengths = _tc_post(end_i32.reshape(1), raw)
    return ids.reshape(B, K, T), lengths.reshape(B, K)


# speedup vs baseline: 392.6779x; 1.6205x over previous
import functools

import jax
import jax.numpy as jnp
from jax import lax
from jax.experimental import pallas as pl
from jax.experimental.pallas import tpu as pltpu
from jax.experimental.pallas import tpu_sc as plsc

T, B, K = 2048, 32, 8
LANES = 16
NCHUNK = 16
CS = T // NCHUNK
W128 = 128
NV = W128 // LANES


def _sc_backtrack_body(step_hbm, par_hbm, osl_hbm, end_hbm, raw_hbm,
                       step_v, par_v, out_v, osl_v, end_v, pub_v, maps_v,
                       maps_sh):
    c = lax.axis_index("c")
    s = lax.axis_index("s")

    pltpu.sync_copy(step_hbm.at[pl.ds(s * CS, CS), pl.ds(c * W128, W128)],
                    step_v)
    pltpu.sync_copy(par_hbm.at[pl.ds(s * CS, CS), pl.ds(c * W128, W128)],
                    par_v)
    pltpu.sync_copy(osl_hbm.at[pl.ds(c * W128, W128)], osl_v)
    pltpu.sync_copy(end_hbm, end_v)

    lane = lax.iota(jnp.int32, LANES)
    kk = lane & (K - 1)
    boffs = [jnp.int32(j * LANES) + (lane & K) for j in range(NV)]
    beams = [jnp.int32(j * LANES) + lane for j in range(NV)]
    svec = jnp.full((LANES,), s, jnp.int32)

    lvec = []
    for j in range(NV):
        m = plsc.load_gather(osl_v, [boffs[j]])
        for jj in range(1, K):
            m = jnp.maximum(m, plsc.load_gather(osl_v, [boffs[j] + jj]))
        lvec.append(jnp.minimum(m, T))

    def clip(w):
        return jnp.minimum(jnp.maximum(w, 0), K - 1)

    def body_a(i, ws):
        tl = CS - 1 - i
        tg = jnp.full((LANES,), s * CS + tl, jnp.int32)
        tlv = jnp.full((LANES,), tl, jnp.int32)
        out = []
        for j in range(NV):
            pv = plsc.load_gather(par_v, [tlv, boffs[j] + clip(ws[j])])
            out.append(jnp.where(tg < lvec[j], pv, ws[j]))
        return tuple(out)

    maps_self = lax.fori_loop(0, CS, body_a, tuple(kk for _ in range(NV)))

    for j in range(NV):
        pub_v[pl.ds(j * LANES, LANES)] = maps_self[j]
    pltpu.sync_copy(pub_v, maps_sh.at[pl.ds(s * W128, W128)])
    plsc.subcore_barrier()
    pltpu.sync_copy(maps_sh, maps_v)

    avec = [kk for _ in range(NV)]
    for u in range(NCHUNK - 1, 0, -1):
        uv = jnp.full((LANES,), u, jnp.int32)
        for j in range(NV):
            au = plsc.load_gather(maps_v,
                                  [jnp.int32(u * W128) + boffs[j]
                                   + clip(avec[j])])
            avec[j] = jnp.where(uv > svec, au, avec[j])

    def body_c(i, ws):
        tl = CS - 1 - i
        tg = jnp.full((LANES,), s * CS + tl, jnp.int32)
        tlv = jnp.full((LANES,), tl, jnp.int32)
        out = []
        for j in range(NV):
            p = boffs[j] + clip(ws[j])
            sv = plsc.load_gather(step_v, [tlv, p])
            pv = plsc.load_gather(par_v, [tlv, p])
            plsc.store_scatter(out_v, [beams[j], tlv], sv)
            out.append(jnp.where(tg < lvec[j], pv, ws[j]))
        return tuple(out)

    lax.fori_loop(0, CS, body_c, tuple(avec))

    base = svec * CS
    for j in range(NV):
        inchunk = (lvec[j] >= base) & (lvec[j] < base + CS)
        plsc.store_scatter(out_v, [beams[j], lvec[j] - base], end_v[...],
                           mask=inchunk)

    pltpu.sync_copy(out_v,
                    raw_hbm.at[pl.ds(c * W128, W128), pl.ds(s * CS, CS)])


_sc_backtrack = functools.partial(
    pl.kernel,
    mesh=plsc.VectorSubcoreMesh(core_axis_name="c", subcore_axis_name="s"),
    out_type=jax.ShapeDtypeStruct((B * K, T), jnp.int32),
    compiler_params=pltpu.CompilerParams(needs_layout_passes=False),
    scratch_types=[
        pltpu.VMEM((CS, W128), jnp.int32),
        pltpu.VMEM((CS, W128), jnp.int32),
        pltpu.VMEM((W128, CS), jnp.int32),
        pltpu.VMEM((W128,), jnp.int32),
        pltpu.VMEM((LANES,), jnp.int32),
        pltpu.VMEM((W128,), jnp.int32),
        pltpu.VMEM((NCHUNK * W128,), jnp.int32),
        pltpu.VMEM_SHARED((NCHUNK * W128,), jnp.int32),
    ],
)(_sc_backtrack_body)

RB = 64


def _tc_post_body(end_ref, raw_ref, ids_ref, len_ref):
    end = end_ref[0]
    raw = raw_ref[...]
    col = lax.broadcasted_iota(jnp.int32, (RB, T), 1)
    cand = jnp.where(raw == end, col, T)
    fe = jnp.min(cand, axis=1, keepdims=True)
    ids_ref[...] = jnp.where(col < fe, raw, end)
    len_ref[...] = fe


def _tc_post(end_arr, raw):
    return pl.pallas_call(
        _tc_post_body,
        grid=(B * K // RB,),
        in_specs=[
            pl.BlockSpec(memory_space=pltpu.SMEM),
            pl.BlockSpec((RB, T), lambda i: (i, 0)),
        ],
        out_specs=[
            pl.BlockSpec((RB, T), lambda i: (i, 0)),
            pl.BlockSpec((RB, 1), lambda i: (i, 0)),
        ],
        out_shape=[
            jax.ShapeDtypeStruct((B * K, T), jnp.int32),
            jax.ShapeDtypeStruct((B * K, 1), jnp.int32),
        ],
    )(end_arr, raw)


def kernel(output_ids, parent_ids, out_seq_lens, end_id):
    osl = out_seq_lens.astype(jnp.int32)
    end_i32 = jnp.asarray(end_id, jnp.int32)
    end_pad = jnp.full((LANES,), end_i32, jnp.int32)
    raw = _sc_backtrack(output_ids, parent_ids, osl, end_pad)
    ids, lengths = _tc_post(end_i32.reshape(1), raw)
    return ids.reshape(B, K, T), lengths.reshape(B, K)
